# baseline (speedup 1.0000x reference)
import functools

import jax
import jax.numpy as jnp
from jax import lax
from jax.experimental import pallas as pl
from jax.experimental.pallas import tpu as pltpu
from jax.experimental.pallas import tpu_sc as plsc

H = 128
NC = 2
NS = 16
NW = NC * NS
C = 64
NSETS = 4


def _emb_body(nchunk, seq, token_hbm, pos_hbm, delta_hbm, segb_hbm, tidx_hbm,
              out_hbm, tix_all, pos_v, delta_v, bufs, gsems, bsems, ssems):
    wid = lax.axis_index("s") * NC + lax.axis_index("c")
    base = wid * (nchunk * C)

    pltpu.sync_copy(pos_hbm, pos_v)
    pltpu.sync_copy(delta_hbm, delta_v)
    pltpu.sync_copy(tidx_hbm.at[wid], tix_all)

    def start_gather(g, s):
        a, _, sb = bufs[s]
        pltpu.async_copy(token_hbm.at[tix_all.at[g]], a, gsems[s])
        pltpu.async_copy(
            segb_hbm.at[pl.ds((base + g * C) * 8, C * 8)], sb, bsems[s])

    def out_slice(g):
        return out_hbm.at[pl.ds(base + g * C, C)]

    def add_chunk(g, s):
        a, o, sb = bufs[s]
        pltpu.make_async_copy(
            segb_hbm.at[pl.ds((base + g * C) * 8, C * 8)], sb,
            bsems[s]).wait()
        pltpu.make_async_copy(token_hbm.at[tix_all.at[g]], a, gsems[s]).wait()
        s_off = lax.rem(base + g * C, seq)
        dv = [delta_v[pl.ds(j * 16, 16)] for j in range(H // 16)]

        @plsc.parallel_loop(0, C // 2, step=1, unroll=2)
        def _(p):
            pair = sb[pl.ds(p * 16, 16)]
            splat0 = lax.bitcast_convert_type(pair << 16, jnp.float32)
            splat1 = lax.bitcast_convert_type(pair & jnp.int32(-65536),
                                              jnp.float32)
            for r, seg_splat in ((2 * p, splat0), (2 * p + 1, splat1)):
                pr = s_off + r
                for j in range(H // 16):
                    sl = pl.ds(j * 16, 16)
                    o[r, sl] = a[r, sl] + pos_v[pr, sl] + seg_splat * dv[j]

    for s in range(NSETS):
        start_gather(s, s)

    def quad(q, carry):
        for s in range(NSETS):
            g = NSETS * q + s
            _, o, _ = bufs[s]

            @pl.when(q > 0)
            def _():
                pltpu.make_async_copy(o, out_slice(g - NSETS),
                                      ssems[s]).wait()

            add_chunk(g, s)

            @pl.when(q < nchunk // NSETS - 1)
            def _():
                start_gather(g + NSETS, s)

            pltpu.async_copy(o, out_slice(g), ssems[s])
        return carry

    lax.fori_loop(0, nchunk // NSETS, quad, 0, unroll=False)

    for s in range(NSETS):
        _, o, _ = bufs[s]
        pltpu.make_async_copy(o, out_slice(nchunk - NSETS + s), ssems[s]).wait()


def kernel(sentences, segments, token_table, segment_table, positional_embedding):
    batch, seq = sentences.shape
    bs = batch * seq
    assert bs % (NW * C) == 0
    nchunk = bs // (NW * C)
    assert nchunk % NSETS == 0

    pos_used = positional_embedding[0, :seq, :]
    pos_ext = (jnp.concatenate([pos_used, pos_used[:C]], axis=0)
               + segment_table[0][None, :])
    delta = segment_table[1] - segment_table[0]
    sp = segments.reshape(NW, nchunk, C // 2, 2) * 0x3F80
    packed = sp[..., 0] + (sp[..., 1] << 16)
    segb = jnp.broadcast_to(
        packed.reshape(NW, nchunk, C // 2, 1).astype(jnp.int32),
        (NW, nchunk, C // 2, 16)).reshape(NW * nchunk * C * 8)
    tidx = sentences.reshape(NW, nchunk, C).astype(jnp.int32)

    mesh = plsc.VectorSubcoreMesh(core_axis_name="c", subcore_axis_name="s")
    run = pl.kernel(
        functools.partial(_emb_body, nchunk, seq),
        out_type=jax.ShapeDtypeStruct((bs, H), jnp.float32),
        mesh=mesh,
        scratch_types=[
            pltpu.VMEM((nchunk, C), jnp.int32),
            pltpu.VMEM((seq + C, H), jnp.float32),
            pltpu.VMEM((H,), jnp.float32),
            tuple(tuple([pltpu.VMEM((C, H), jnp.float32),
                         pltpu.VMEM((C, H), jnp.float32),
                         pltpu.VMEM((C * 8,), jnp.int32)])
                  for _ in range(NSETS)),
            tuple(pltpu.SemaphoreType.DMA for _ in range(NSETS)),
            tuple(pltpu.SemaphoreType.DMA for _ in range(NSETS)),
            tuple(pltpu.SemaphoreType.DMA for _ in range(NSETS)),
        ],
    )
    out = run(token_table, pos_ext, delta, segb, tidx)
    return out.reshape(batch, seq, H)


# speedup vs baseline: 2.5921x; 2.5921x over previous
import functools

import jax
import jax.numpy as jnp
from jax import lax
from jax.experimental import pallas as pl
from jax.experimental.pallas import tpu as pltpu
from jax.experimental.pallas import tpu_sc as plsc

H = 128
NC = 2
NS = 16
NW = NC * NS
C = 64
NSETS = 4


def _emb_body(nchunk, seq, token_hbm, pos_hbm, delta_hbm, segb_hbm, tidx_hbm,
              out_hbm, tix_all, pos_v, delta_v, bufs, gsems, bsems, ssems):
    wid = lax.axis_index("s") * NC + lax.axis_index("c")
    base = wid * (nchunk * C)

    pltpu.sync_copy(pos_hbm, pos_v)
    pltpu.sync_copy(delta_hbm, delta_v)
    pltpu.sync_copy(tidx_hbm.at[wid], tix_all)

    def start_gather(g, s):
        a, _, sb = bufs[s]
        pltpu.async_copy(token_hbm.at[tix_all.at[g]], a, gsems[s])
        pltpu.async_copy(segb_hbm.at[wid].at[g], sb, bsems[s])

    def out_slice(g):
        return out_hbm.at[pl.ds(base + g * C, C)]

    def add_chunk(g, s):
        a, o, sb = bufs[s]
        pltpu.make_async_copy(segb_hbm.at[wid].at[g], sb, bsems[s]).wait()
        pltpu.make_async_copy(token_hbm.at[tix_all.at[g]], a, gsems[s]).wait()
        s_off = lax.rem(base + g * C, seq)
        dv = [delta_v[pl.ds(j * 16, 16)] for j in range(H // 16)]

        @plsc.parallel_loop(0, C, step=1, unroll=4)
        def _(r):
            seg_splat = sb[pl.ds(r * 16, 16)]
            pr = s_off + r
            for j in range(H // 16):
                sl = pl.ds(j * 16, 16)
                o[r, sl] = a[r, sl] + pos_v[pr, sl] + seg_splat * dv[j]

    for s in range(NSETS):
        start_gather(s, s)

    def quad(q, carry):
        for s in range(NSETS):
            g = NSETS * q + s
            _, o, _ = bufs[s]

            @pl.when(q > 0)
            def _():
                pltpu.make_async_copy(o, out_slice(g - NSETS),
                                      ssems[s]).wait()

            add_chunk(g, s)

            @pl.when(q < nchunk // NSETS - 1)
            def _():
                start_gather(g + NSETS, s)

            pltpu.async_copy(o, out_slice(g), ssems[s])
        return carry

    lax.fori_loop(0, nchunk // NSETS, quad, 0, unroll=False)

    for s in range(NSETS):
        _, o, _ = bufs[s]
        pltpu.make_async_copy(o, out_slice(nchunk - NSETS + s), ssems[s]).wait()


def kernel(sentences, segments, token_table, segment_table, positional_embedding):
    batch, seq = sentences.shape
    bs = batch * seq
    assert bs % (NW * C) == 0
    nchunk = bs // (NW * C)
    assert nchunk % NSETS == 0

    pos_used = positional_embedding[0, :seq, :]
    pos_ext = (jnp.concatenate([pos_used, pos_used[:C]], axis=0)
               + segment_table[0][None, :])
    delta = segment_table[1] - segment_table[0]
    segb = jnp.broadcast_to(
        segments.reshape(NW, nchunk, C, 1).astype(jnp.float32),
        (NW, nchunk, C, 16)).reshape(NW, nchunk, C * 16)
    tidx = sentences.reshape(NW, nchunk, C).astype(jnp.int32)

    mesh = plsc.VectorSubcoreMesh(core_axis_name="c", subcore_axis_name="s")
    run = pl.kernel(
        functools.partial(_emb_body, nchunk, seq),
        out_type=jax.ShapeDtypeStruct((bs, H), jnp.float32),
        mesh=mesh,
        scratch_types=[
            pltpu.VMEM((nchunk, C), jnp.int32),
            pltpu.VMEM((seq + C, H), jnp.float32),
            pltpu.VMEM((H,), jnp.float32),
            tuple(tuple([pltpu.VMEM((C, H), jnp.float32),
                         pltpu.VMEM((C, H), jnp.float32),
                         pltpu.VMEM((C * 16,), jnp.float32)])
                  for _ in range(NSETS)),
            tuple(pltpu.SemaphoreType.DMA for _ in range(NSETS)),
            tuple(pltpu.SemaphoreType.DMA for _ in range(NSETS)),
            tuple(pltpu.SemaphoreType.DMA for _ in range(NSETS)),
        ],
    )
    out = run(token_table, pos_ext, delta, segb, tidx)
    return out.reshape(batch, seq, H)
